# baseline (speedup 1.0000x reference)
import jax
import jax.numpy as jnp
from jax import lax
from jax.experimental import pallas as pl
from jax.experimental.pallas import tpu as pltpu
from jax.experimental.pallas import tpu_sc as plsc

_NC = 2
_NS = 16
_L = 16

_C = 96
_NBUF = 4
_W0 = 128


def _sc_gather(idx_hbm, x_hbm, out_hbm, idxf_v, sel_v, bufs, gsems, ssems,
               slabs_per_w, h, w):
    cid = lax.axis_index("c")
    sid = lax.axis_index("s")
    wid = sid * _NC + cid

    pltpu.sync_copy(idx_hbm, idxf_v)
    zeros = jnp.zeros((_L,), jnp.int32)
    for k in range(_C // _L):
        sel_v[pl.ds(_L * k, _L)] = zeros
    iota = lax.iota(jnp.int32, _L)
    ones = jnp.ones((_L,), jnp.int32)
    offset = zeros
    for k in range(_C // _L):
        v = idxf_v[pl.ds(_L * k, _L)]
        m = v != jnp.zeros((_L,), jnp.float32)
        mi = jnp.where(m, ones, zeros)
        pos = plsc.cumsum(mi) - ones + offset
        plsc.store_scatter(sel_v, [pos], iota + jnp.full((_L,), _L * k, jnp.int32), mask=m)
        offset = offset + lax.broadcast(jnp.sum(mi), (_L,))

    base_slab = wid * slabs_per_w
    col_off = (0, _W0)
    col_len = (_W0, w - _W0)

    def src_dst(slab, col):
        s_global = base_slab + slab
        bi = s_global // _C
        j = s_global - bi * _C
        sel_vec = plsc.load_gather(sel_v, [lax.broadcast(j, (_L,))])
        sj = jnp.max(sel_vec)
        wo, wl = col_off[col], col_len[col]
        return (x_hbm.at[pl.ds(bi, 1), pl.ds(sj, 1), pl.ds(0, h), pl.ds(wo, wl)],
                out_hbm.at[pl.ds(bi, 1), pl.ds(j, 1), pl.ds(0, h), pl.ds(wo, wl)])

    def outer(go, carry):
        for b in range(_NBUF):
            slab = go * (_NBUF // 2) + b // 2
            src, _ = src_dst(slab, b % 2)

            @pl.when(go > 0)
            def _():
                pltpu.make_async_copy(
                    bufs[b],
                    out_hbm.at[pl.ds(0, 1), pl.ds(0, 1), pl.ds(0, h),
                               pl.ds(col_off[b % 2], col_len[b % 2])],
                    ssems[b],
                ).wait()

            pltpu.async_copy(src, bufs[b], gsems[b])
        for b in range(_NBUF):
            slab = go * (_NBUF // 2) + b // 2
            _, dst = src_dst(slab, b % 2)
            pltpu.make_async_copy(
                x_hbm.at[pl.ds(0, 1), pl.ds(0, 1), pl.ds(0, h),
                         pl.ds(col_off[b % 2], col_len[b % 2])],
                bufs[b], gsems[b],
            ).wait()
            pltpu.async_copy(bufs[b], dst, ssems[b])
        return carry

    lax.fori_loop(0, 2 * slabs_per_w // _NBUF, outer, jnp.int32(0))
    for b in range(_NBUF):
        pltpu.make_async_copy(
            bufs[b],
            out_hbm.at[pl.ds(0, 1), pl.ds(0, 1), pl.ds(0, h),
                       pl.ds(col_off[b % 2], col_len[b % 2])],
            ssems[b],
        ).wait()


@jax.jit
def kernel(input_tensor, indexes):
    b, c, h, w = input_tensor.shape
    n_slabs = b * c
    n_workers = _NC * _NS
    slabs_per_w = n_slabs // n_workers

    mesh = plsc.VectorSubcoreMesh(
        core_axis_name="c", subcore_axis_name="s",
        num_cores=_NC, num_subcores=_NS,
    )

    def body(idx_hbm, x_hbm, out_hbm, idxf_v, sel_v, *rest):
        _sc_gather(idx_hbm, x_hbm, out_hbm, idxf_v, sel_v,
                   list(rest[0:_NBUF]), list(rest[_NBUF:2 * _NBUF]),
                   list(rest[2 * _NBUF:3 * _NBUF]), slabs_per_w, h, w)

    buf_types = []
    for i in range(_NBUF):
        wl = _W0 if i % 2 == 0 else w - _W0
        buf_types.append(pltpu.VMEM((1, 1, h, wl), jnp.float32))

    return pl.kernel(
        body,
        out_type=jax.ShapeDtypeStruct((b, c, h, w), jnp.float32),
        mesh=mesh,
        compiler_params=pltpu.CompilerParams(
            needs_layout_passes=False, use_tc_tiling_on_sc=True,
        ),
        scratch_types=(
            [pltpu.VMEM((c,), jnp.float32), pltpu.VMEM((c,), jnp.int32)]
            + buf_types
            + [pltpu.SemaphoreType.DMA] * (2 * _NBUF)
        ),
    )(indexes, input_tensor)


# speedup vs baseline: 1.0205x; 1.0205x over previous
import jax
import jax.numpy as jnp
from jax import lax
from jax.experimental import pallas as pl
from jax.experimental.pallas import tpu as pltpu
from jax.experimental.pallas import tpu_sc as plsc

_NC = 2
_NS = 16
_L = 16

_C = 96
_NBUF = 8


def _sc_gather(idx_hbm, x_hbm, out_hbm, idxf_v, sel_v, bufs, gsems, ssems,
               slabs_per_w, h, w):
    cid = lax.axis_index("c")
    sid = lax.axis_index("s")
    wid = sid * _NC + cid

    pltpu.sync_copy(idx_hbm, idxf_v)
    zeros = jnp.zeros((_L,), jnp.int32)
    for k in range(_C // _L):
        sel_v[pl.ds(_L * k, _L)] = zeros
    iota = lax.iota(jnp.int32, _L)
    ones = jnp.ones((_L,), jnp.int32)
    offset = zeros
    for k in range(_C // _L):
        v = idxf_v[pl.ds(_L * k, _L)]
        m = v != jnp.zeros((_L,), jnp.float32)
        mi = jnp.where(m, ones, zeros)
        pos = plsc.cumsum(mi) - ones + offset
        plsc.store_scatter(sel_v, [pos], iota + jnp.full((_L,), _L * k, jnp.int32), mask=m)
        offset = offset + lax.broadcast(jnp.sum(mi), (_L,))

    base_slab = wid * slabs_per_w
    hh = h // 4

    def src_dst(t):
        s_global = base_slab + t // 4
        half = t - (t // 4) * 4
        bi = s_global // _C
        j = s_global - bi * _C
        sel_vec = plsc.load_gather(sel_v, [lax.broadcast(j, (_L,))])
        sj = jnp.max(sel_vec)
        ro = half * hh
        return (x_hbm.at[pl.ds(bi, 1), pl.ds(sj, 1), pl.ds(ro, hh)],
                out_hbm.at[pl.ds(bi, 1), pl.ds(j, 1), pl.ds(ro, hh)])

    def outer(go, carry):
        for b in range(_NBUF):
            t = go * _NBUF + b
            src, _ = src_dst(t)

            @pl.when(go > 0)
            def _():
                pltpu.make_async_copy(
                    bufs[b], out_hbm.at[pl.ds(0, 1), pl.ds(0, 1), pl.ds(0, h // 4)], ssems[b]
                ).wait()

            pltpu.async_copy(src, bufs[b], gsems[b])
        for b in range(_NBUF):
            t = go * _NBUF + b
            _, dst = src_dst(t)
            pltpu.make_async_copy(
                x_hbm.at[pl.ds(0, 1), pl.ds(0, 1), pl.ds(0, h // 4)], bufs[b], gsems[b]
            ).wait()
            pltpu.async_copy(bufs[b], dst, ssems[b])
        return carry

    lax.fori_loop(0, 4 * slabs_per_w // _NBUF, outer, jnp.int32(0))
    for b in range(_NBUF):
        pltpu.make_async_copy(
            bufs[b], out_hbm.at[pl.ds(0, 1), pl.ds(0, 1), pl.ds(0, h // 4)], ssems[b]
        ).wait()


@jax.jit
def kernel(input_tensor, indexes):
    b, c, h, w = input_tensor.shape
    n_slabs = b * c
    n_workers = _NC * _NS
    slabs_per_w = n_slabs // n_workers

    mesh = plsc.VectorSubcoreMesh(
        core_axis_name="c", subcore_axis_name="s",
        num_cores=_NC, num_subcores=_NS,
    )

    def body(idx_hbm, x_hbm, out_hbm, idxf_v, sel_v, *rest):
        _sc_gather(idx_hbm, x_hbm, out_hbm, idxf_v, sel_v,
                   list(rest[0:_NBUF]), list(rest[_NBUF:2 * _NBUF]),
                   list(rest[2 * _NBUF:3 * _NBUF]), slabs_per_w, h, w)

    return pl.kernel(
        body,
        out_type=jax.ShapeDtypeStruct((b, c, h, w), jnp.float32),
        mesh=mesh,
        compiler_params=pltpu.CompilerParams(
            needs_layout_passes=False, use_tc_tiling_on_sc=True,
        ),
        scratch_types=(
            [pltpu.VMEM((c,), jnp.float32), pltpu.VMEM((c,), jnp.int32)]
            + [pltpu.VMEM((1, 1, h // 4, w), jnp.float32)] * _NBUF
            + [pltpu.SemaphoreType.DMA] * (2 * _NBUF)
        ),
    )(indexes, input_tensor)
